# speedup vs baseline: 9.8860x; 9.8860x over previous
import math

import jax
import jax.numpy as jnp
from jax.experimental import pallas as pl
from jax.experimental.pallas import tpu as pltpu

_SMOOTHING = 0.1
_VOCAB = 32000
_N_TOKENS = 2048
_CONF = 1.0 - _SMOOTHING
_SVAL = _SMOOTHING / float(_VOCAB - 2)
_C_HIT = (_VOCAB - 2) * _SVAL * math.log(_SVAL) + _CONF * math.log(_CONF)
_C_IGN = (_VOCAB - 1) * _SVAL * math.log(_SVAL) + _CONF * math.log(_CONF)

_BC = 1280
_NBLK = _VOCAB // _BC


def _tc_body(x_ref, tgt_ref, out_ref, acc_s, acc_t, acc_z):
    j = pl.program_id(0)

    @pl.when(j == 0)
    def _init():
        acc_s[...] = jnp.zeros_like(acc_s)
        acc_t[...] = jnp.zeros_like(acc_t)
        acc_z[...] = x_ref[:, 0:1]

    x = x_ref[...]
    acc_s[...] += jnp.sum(x, axis=1, keepdims=True)
    cols = jax.lax.broadcasted_iota(jnp.int32, x.shape, 1) + j * _BC
    hit = cols == tgt_ref[...]
    acc_t[...] += jnp.sum(jnp.where(hit, x, 0.0), axis=1, keepdims=True)

    @pl.when(j == _NBLK - 1)
    def _fin():
        s = acc_s[...]
        t = acc_t[...]
        z = acc_z[...]
        tgt = tgt_ref[...]
        hit_val = _C_HIT + _SVAL * z + (_SVAL - _CONF) * t
        ign_val = _C_IGN + (_SVAL - _CONF) * z
        out_ref[...] = jnp.where(tgt == 0, ign_val, hit_val) - _SVAL * s


def _tc_call(x, tgt2d, interpret=False):
    return pl.pallas_call(
        _tc_body,
        grid=(_NBLK,),
        in_specs=[
            pl.BlockSpec((_N_TOKENS, _BC), lambda j: (0, j)),
            pl.BlockSpec((_N_TOKENS, 1), lambda j: (0, 0)),
        ],
        out_specs=pl.BlockSpec((_N_TOKENS, 1), lambda j: (0, 0)),
        out_shape=jax.ShapeDtypeStruct((_N_TOKENS, 1), jnp.float32),
        scratch_shapes=[
            pltpu.VMEM((_N_TOKENS, 1), jnp.float32),
            pltpu.VMEM((_N_TOKENS, 1), jnp.float32),
            pltpu.VMEM((_N_TOKENS, 1), jnp.float32),
        ],
        interpret=interpret,
    )(x, tgt2d)


def kernel(model_prob, target):
    tgt2d = target.astype(jnp.int32).reshape(_N_TOKENS, 1)
    out = _tc_call(model_prob, tgt2d)
    return out[:, 0]
